# baseline (speedup 1.0000x reference)
import functools

import jax
import jax.numpy as jnp
from jax import lax
from jax.experimental import pallas as pl
from jax.experimental.pallas import tpu as pltpu
from jax.experimental.pallas import tpu_sc as plsc

_N = 10000
_E = 320000
_D = 128

_NC = 2
_NS = 16
_NW = _NC * _NS
_CH = 80
_NCHUNK = 125
_EPT = _CH * _NCHUNK
_ROWS = 10240
_RPS = _ROWS // _NS

def _zero_fill(buf, nrow):
    def row(i, _):
        for j in range(_D // 16):
            buf[i, pl.ds(j * 16, 16)] = jnp.zeros((16,), jnp.float32)
        return _
    lax.fori_loop(0, nrow, row, None)


def _deg_body(dst_hbm, out_hbm, *sc):
    dstb = sc[0:8]
    ones_v = sc[8]
    zb_v = sc[9]
    acc = sc[10]
    lsem = sc[11:19]
    ssem = sc[19:27]

    c = lax.axis_index("c")
    s = lax.axis_index("s")
    wid = c * _NS + s
    base = wid * _EPT

    def dload(j, w):
        pltpu.async_copy(dst_hbm.at[pl.ds(base + j * _CH, _CH)], dstb[w], lsem[w])

    def dload_wait(w):
        pltpu.make_async_copy(dst_hbm.at[pl.ds(base, _CH)], dstb[w], lsem[w]).wait()

    def dscat(w):
        pltpu.async_copy(ones_v, acc.at[dstb[w]], ssem[w], add=True)

    def dscat_wait(w):
        pltpu.make_async_copy(ones_v, acc.at[dstb[0]], ssem[w]).wait()

    def fill_ones(i, _):
        ones_v[pl.ds(i * 16, 16)] = jnp.ones((16,), jnp.float32)
        return _
    lax.fori_loop(0, _CH // 16, fill_ones, None)

    def fill_zero(i, _):
        zb_v[pl.ds(i * 16, 16)] = jnp.zeros((16,), jnp.float32)
        return _
    lax.fori_loop(0, _RPS // 16, fill_zero, None)

    for w in range(8):
        dload(w, w)
    pltpu.sync_copy(zb_v, acc.at[pl.ds(s * _RPS, _RPS)])
    plsc.subcore_barrier()

    def dstep(j, t, reload=True):
        x = t % 8
        dload_wait(x)
        dscat(x)
        if reload:
            dscat_wait((t + 4) % 8)
            dload(j + 4, (t + 4) % 8)

    for j in range(4):
        dstep(j, j, reload=False)

    def dgroup(g, _):
        for t8 in range(8):
            dstep(8 * g + 4 + t8, 4 + t8)
        return _
    lax.fori_loop(0, 14, dgroup, None)

    for j in range(116, 121):
        dstep(j, j % 8)
    for j in range(121, 125):
        dstep(j, j % 8, reload=False)
    for j in range(117, 125):
        dscat_wait(j % 8)

    plsc.subcore_barrier()
    pltpu.sync_copy(acc.at[pl.ds(s * _RPS, _RPS)],
                    out_hbm.at[c, pl.ds(s * _RPS, _RPS)])


def _agg_body(y_hbm, src_hbm, dst_hbm, out_hbm, *sc):
    rows = sc[0:4]
    srcb = sc[4:12]
    dstb = sc[12:20]
    acc = sc[20]
    gsem = sc[21:25]
    ssem = sc[25:29]
    lsem = sc[29:37]
    zsem = sc[37]

    c = lax.axis_index("c")
    s = lax.axis_index("s")
    wid = c * _NS + s
    base = wid * _EPT

    def load_start(j, w):
        pltpu.async_copy(src_hbm.at[pl.ds(base + j * _CH, _CH)], srcb[w], lsem[w])
        pltpu.async_copy(dst_hbm.at[pl.ds(base + j * _CH, _CH)], dstb[w], lsem[w])

    def load_wait(w):
        pltpu.make_async_copy(src_hbm.at[pl.ds(base, _CH)], srcb[w], lsem[w]).wait()
        pltpu.make_async_copy(dst_hbm.at[pl.ds(base, _CH)], dstb[w], lsem[w]).wait()

    def gather_start(x, w):
        pltpu.async_copy(y_hbm.at[srcb[w]], rows[x], gsem[x])

    def gather_wait(x):
        pltpu.make_async_copy(y_hbm.at[srcb[0]], rows[x], gsem[x]).wait()

    def scat_start(x, w):
        pltpu.async_copy(rows[x], acc.at[dstb[w]], ssem[x], add=True)

    def scat_wait(x):
        pltpu.make_async_copy(rows[0], acc.at[dstb[0]], ssem[x]).wait()

    _zero_fill(rows[3], _CH)
    zd = [pltpu.async_copy(rows[3], acc.at[pl.ds(s * _RPS + i * _CH, _CH)], zsem)
          for i in range(_RPS // _CH)]
    for w in range(5):
        load_start(w, w)
    load_wait(0)
    gather_start(0, 0)
    load_wait(1)
    gather_start(1, 1)
    load_wait(2)
    gather_start(2, 2)
    for d in zd:
        d.wait()
    plsc.subcore_barrier()

    def step(j, t, s_wait=True, g_issue=True, l_issue=True):
        x = t % 4
        gather_wait(x)
        scat_start(x, t % 8)
        if g_issue:
            if s_wait:
                scat_wait((t + 3) % 4)
            load_wait((t + 3) % 8)
            gather_start((t + 3) % 4, (t + 3) % 8)
        if l_issue:
            load_start(j + 5, (t + 5) % 8)

    step(0, 0, s_wait=False)
    step(1, 1)
    step(2, 2)

    def group(g, _):
        for t8 in range(8):
            step(8 * g + 3 + t8, 3 + t8)
        return _
    lax.fori_loop(0, 14, group, None)

    for j in range(115, 120):
        step(j, j % 8)
    step(120, 120 % 8, l_issue=False)
    step(121, 121 % 8, l_issue=False)
    step(122, 122 % 8, g_issue=False, l_issue=False)
    step(123, 123 % 8, g_issue=False, l_issue=False)
    step(124, 124 % 8, g_issue=False, l_issue=False)
    for x in range(4):
        scat_wait((121 + x) % 4)

    plsc.subcore_barrier()
    pltpu.sync_copy(acc.at[pl.ds(s * _RPS, _RPS)],
                    out_hbm.at[c, pl.ds(s * _RPS, _RPS)])


@functools.lru_cache(maxsize=None)
def _sc_kernels():
    mesh = plsc.VectorSubcoreMesh(core_axis_name="c", subcore_axis_name="s",
                                  num_cores=_NC, num_subcores=_NS)
    deg = pl.kernel(
        _deg_body,
        out_type=jax.ShapeDtypeStruct((_NC, _ROWS), jnp.float32),
        mesh=mesh,
        scratch_types=(
            [pltpu.VMEM((_CH,), jnp.int32) for _ in range(8)]
            + [pltpu.VMEM((_CH,), jnp.float32)]
            + [pltpu.VMEM((_RPS,), jnp.float32)]
            + [pltpu.VMEM_SHARED((_ROWS,), jnp.float32)]
            + [pltpu.SemaphoreType.DMA for _ in range(16)]
        ),
    )
    agg = pl.kernel(
        _agg_body,
        out_type=jax.ShapeDtypeStruct((_NC, _ROWS, _D), jnp.float32),
        mesh=mesh,
        scratch_types=(
            [pltpu.VMEM((_CH, _D), jnp.float32) for _ in range(4)]
            + [pltpu.VMEM((_CH,), jnp.int32) for _ in range(8)]
            + [pltpu.VMEM((_CH,), jnp.int32) for _ in range(8)]
            + [pltpu.VMEM_SHARED((_ROWS, _D), jnp.float32)]
            + [pltpu.SemaphoreType.DMA for _ in range(17)]
        ),
    )
    return deg, agg


def _dis_from(dp_ref):
    deg = dp_ref[0, :] + dp_ref[1, :] + 1.0
    return lax.rsqrt(deg)[:_N]


def _prep_body(x_ref, w_ref, dp_ref, y_ref):
    dis = _dis_from(dp_ref)
    xw = jnp.dot(x_ref[...], w_ref[...], preferred_element_type=jnp.float32)
    y_ref[...] = xw * dis[:, None]


def _mid_body(p_ref, y_ref, dp_ref, b_ref, w_ref, o_ref):
    dis = _dis_from(dp_ref)
    agg = p_ref[0, :_N, :] + p_ref[1, :_N, :]
    h = jnp.maximum(dis[:, None] * (agg + y_ref[...]) + b_ref[...][None, :], 0.0)
    hw = jnp.dot(h, w_ref[...], preferred_element_type=jnp.float32)
    o_ref[...] = hw * dis[:, None]


def _fin_body(p_ref, y_ref, dp_ref, b_ref, o_ref):
    dis = _dis_from(dp_ref)
    agg = p_ref[0, :_N, :] + p_ref[1, :_N, :]
    o_ref[...] = jnp.maximum(
        dis[:, None] * (agg + y_ref[...]) + b_ref[...][None, :], 0.0)


_f32 = jnp.float32
_prep_call = pl.pallas_call(
    _prep_body, out_shape=jax.ShapeDtypeStruct((_N, _D), _f32))
_mid_call = pl.pallas_call(
    _mid_body, out_shape=jax.ShapeDtypeStruct((_N, _D), _f32))
_fin_call = pl.pallas_call(
    _fin_body, out_shape=jax.ShapeDtypeStruct((_N, _D), _f32))


def kernel(x, edge_index, W1, b1, W2, b2):
    src_p = edge_index[0].astype(jnp.int32)
    dst_p = edge_index[1].astype(jnp.int32)

    deg_kernel, agg_kernel = _sc_kernels()
    dp = deg_kernel(dst_p)
    y1 = _prep_call(x, W1, dp)
    p1 = agg_kernel(y1, src_p, dst_p)
    y2 = _mid_call(p1, y1, dp, b1, W2)
    p2 = agg_kernel(y2, src_p, dst_p)
    return _fin_call(p2, y2, dp, b2)


# speedup vs baseline: 1.0443x; 1.0443x over previous
import functools

import jax
import jax.numpy as jnp
from jax import lax
from jax.experimental import pallas as pl
from jax.experimental.pallas import tpu as pltpu
from jax.experimental.pallas import tpu_sc as plsc

_N = 10000
_E = 320000
_D = 128

_NC = 2
_NS = 16
_NW = _NC * _NS
_CH = 80
_NCHUNK = 125
_EPT = _CH * _NCHUNK
_ROWS = 10240
_RPS = _ROWS // _NS

def _zero_fill(buf, nrow):
    def row(i, _):
        for j in range(_D // 16):
            buf[i, pl.ds(j * 16, 16)] = jnp.zeros((16,), jnp.float32)
        return _
    lax.fori_loop(0, nrow, row, None)


def _deg_body(edges_hbm, out_hbm, *sc):
    dstb = sc[0:8]
    ones_v = sc[8]
    zb_v = sc[9]
    acc = sc[10]
    lsem = sc[11:19]
    ssem = sc[19:27]

    c = lax.axis_index("c")
    s = lax.axis_index("s")
    wid = c * _NS + s
    base = wid * _EPT

    def dload(j, w):
        pltpu.async_copy(edges_hbm.at[pl.ds(_E + base + j * _CH, _CH)],
                         dstb[w], lsem[w])

    def dload_wait(w):
        pltpu.make_async_copy(edges_hbm.at[pl.ds(_E + base, _CH)],
                              dstb[w], lsem[w]).wait()

    def dscat(w):
        pltpu.async_copy(ones_v, acc.at[dstb[w]], ssem[w], add=True)

    def dscat_wait(w):
        pltpu.make_async_copy(ones_v, acc.at[dstb[0]], ssem[w]).wait()

    def fill_ones(i, _):
        ones_v[pl.ds(i * 16, 16)] = jnp.ones((16,), jnp.float32)
        return _
    lax.fori_loop(0, _CH // 16, fill_ones, None)

    def fill_zero(i, _):
        zb_v[pl.ds(i * 16, 16)] = jnp.zeros((16,), jnp.float32)
        return _
    lax.fori_loop(0, _RPS // 16, fill_zero, None)

    for w in range(8):
        dload(w, w)
    pltpu.sync_copy(zb_v, acc.at[pl.ds(s * _RPS, _RPS)])
    plsc.subcore_barrier()

    def dstep(j, t, reload=True):
        x = t % 8
        dload_wait(x)
        dscat(x)
        if reload:
            dscat_wait((t + 4) % 8)
            dload(j + 4, (t + 4) % 8)

    for j in range(4):
        dstep(j, j, reload=False)

    def dgroup(g, _):
        for t8 in range(8):
            dstep(8 * g + 4 + t8, 4 + t8)
        return _
    lax.fori_loop(0, 14, dgroup, None)

    for j in range(116, 121):
        dstep(j, j % 8)
    for j in range(121, 125):
        dstep(j, j % 8, reload=False)
    for j in range(117, 125):
        dscat_wait(j % 8)

    plsc.subcore_barrier()
    pltpu.sync_copy(acc.at[pl.ds(s * _RPS, _RPS)],
                    out_hbm.at[c, pl.ds(s * _RPS, _RPS)])


def _agg_body(y_hbm, edges_hbm, out_hbm, *sc):
    rows = sc[0:4]
    srcb = sc[4:12]
    dstb = sc[12:20]
    acc = sc[20]
    gsem = sc[21:25]
    ssem = sc[25:29]
    lsem = sc[29:37]
    zsem = sc[37]

    c = lax.axis_index("c")
    s = lax.axis_index("s")
    wid = c * _NS + s
    base = wid * _EPT

    def load_start(j, w):
        pltpu.async_copy(edges_hbm.at[pl.ds(base + j * _CH, _CH)],
                         srcb[w], lsem[w])
        pltpu.async_copy(edges_hbm.at[pl.ds(_E + base + j * _CH, _CH)],
                         dstb[w], lsem[w])

    def load_wait(w):
        pltpu.make_async_copy(edges_hbm.at[pl.ds(base, _CH)],
                              srcb[w], lsem[w]).wait()
        pltpu.make_async_copy(edges_hbm.at[pl.ds(base, _CH)],
                              dstb[w], lsem[w]).wait()

    def gather_start(x, w):
        pltpu.async_copy(y_hbm.at[srcb[w]], rows[x], gsem[x])

    def gather_wait(x):
        pltpu.make_async_copy(y_hbm.at[srcb[0]], rows[x], gsem[x]).wait()

    def scat_start(x, w):
        pltpu.async_copy(rows[x], acc.at[dstb[w]], ssem[x], add=True)

    def scat_wait(x):
        pltpu.make_async_copy(rows[0], acc.at[dstb[0]], ssem[x]).wait()

    _zero_fill(rows[3], _CH)
    zd = [pltpu.async_copy(rows[3], acc.at[pl.ds(s * _RPS + i * _CH, _CH)], zsem)
          for i in range(_RPS // _CH)]
    for w in range(5):
        load_start(w, w)
    load_wait(0)
    gather_start(0, 0)
    load_wait(1)
    gather_start(1, 1)
    load_wait(2)
    gather_start(2, 2)
    for d in zd:
        d.wait()
    plsc.subcore_barrier()

    def step(j, t, s_wait=True, g_issue=True, l_issue=True):
        x = t % 4
        gather_wait(x)
        scat_start(x, t % 8)
        if g_issue:
            if s_wait:
                scat_wait((t + 3) % 4)
            load_wait((t + 3) % 8)
            gather_start((t + 3) % 4, (t + 3) % 8)
        if l_issue:
            load_start(j + 5, (t + 5) % 8)

    step(0, 0, s_wait=False)
    step(1, 1)
    step(2, 2)

    def group(g, _):
        for t8 in range(8):
            step(8 * g + 3 + t8, 3 + t8)
        return _
    lax.fori_loop(0, 14, group, None)

    for j in range(115, 120):
        step(j, j % 8)
    step(120, 120 % 8, l_issue=False)
    step(121, 121 % 8, l_issue=False)
    step(122, 122 % 8, g_issue=False, l_issue=False)
    step(123, 123 % 8, g_issue=False, l_issue=False)
    step(124, 124 % 8, g_issue=False, l_issue=False)
    for x in range(4):
        scat_wait((121 + x) % 4)

    plsc.subcore_barrier()
    pltpu.sync_copy(acc.at[pl.ds(s * _RPS, _RPS)],
                    out_hbm.at[c, pl.ds(s * _RPS, _RPS)])


@functools.lru_cache(maxsize=None)
def _sc_kernels():
    mesh = plsc.VectorSubcoreMesh(core_axis_name="c", subcore_axis_name="s",
                                  num_cores=_NC, num_subcores=_NS)
    deg = pl.kernel(
        _deg_body,
        out_type=jax.ShapeDtypeStruct((_NC, _ROWS), jnp.float32),
        mesh=mesh,
        scratch_types=(
            [pltpu.VMEM((_CH,), jnp.int32) for _ in range(8)]
            + [pltpu.VMEM((_CH,), jnp.float32)]
            + [pltpu.VMEM((_RPS,), jnp.float32)]
            + [pltpu.VMEM_SHARED((_ROWS,), jnp.float32)]
            + [pltpu.SemaphoreType.DMA for _ in range(16)]
        ),
    )
    agg = pl.kernel(
        _agg_body,
        out_type=jax.ShapeDtypeStruct((_NC, _ROWS, _D), jnp.float32),
        mesh=mesh,
        scratch_types=(
            [pltpu.VMEM((_CH, _D), jnp.float32) for _ in range(4)]
            + [pltpu.VMEM((_CH,), jnp.int32) for _ in range(8)]
            + [pltpu.VMEM((_CH,), jnp.int32) for _ in range(8)]
            + [pltpu.VMEM_SHARED((_ROWS, _D), jnp.float32)]
            + [pltpu.SemaphoreType.DMA for _ in range(17)]
        ),
    )
    return deg, agg


def _dis_from(dp_ref):
    deg = dp_ref[0, :] + dp_ref[1, :] + 1.0
    return lax.rsqrt(deg)[:_N]


def _prep_body(x_ref, w_ref, dp_ref, y_ref):
    dis = _dis_from(dp_ref)
    xw = jnp.dot(x_ref[...], w_ref[...], preferred_element_type=jnp.float32)
    y_ref[...] = xw * dis[:, None]


def _mid_body(p_ref, y_ref, dp_ref, b_ref, w_ref, o_ref):
    dis = _dis_from(dp_ref)
    agg = p_ref[0, :_N, :] + p_ref[1, :_N, :]
    h = jnp.maximum(dis[:, None] * (agg + y_ref[...]) + b_ref[...][None, :], 0.0)
    hw = jnp.dot(h, w_ref[...], preferred_element_type=jnp.float32)
    o_ref[...] = hw * dis[:, None]


def _fin_body(p_ref, y_ref, dp_ref, b_ref, o_ref):
    dis = _dis_from(dp_ref)
    agg = p_ref[0, :_N, :] + p_ref[1, :_N, :]
    o_ref[...] = jnp.maximum(
        dis[:, None] * (agg + y_ref[...]) + b_ref[...][None, :], 0.0)


_f32 = jnp.float32
_prep_call = pl.pallas_call(
    _prep_body, out_shape=jax.ShapeDtypeStruct((_N, _D), _f32))
_mid_call = pl.pallas_call(
    _mid_body, out_shape=jax.ShapeDtypeStruct((_N, _D), _f32))
_fin_call = pl.pallas_call(
    _fin_body, out_shape=jax.ShapeDtypeStruct((_N, _D), _f32))


def kernel(x, edge_index, W1, b1, W2, b2):
    edges = edge_index.astype(jnp.int32).reshape(2 * _E)

    deg_kernel, agg_kernel = _sc_kernels()
    dp = deg_kernel(edges)
    y1 = _prep_call(x, W1, dp)
    p1 = agg_kernel(y1, edges)
    y2 = _mid_call(p1, y1, dp, b1, W2)
    p2 = agg_kernel(y2, edges)
    return _fin_call(p2, y2, dp, b2)


# speedup vs baseline: 1.0887x; 1.0425x over previous
import functools

import jax
import jax.numpy as jnp
from jax import lax
from jax.experimental import pallas as pl
from jax.experimental.pallas import tpu as pltpu
from jax.experimental.pallas import tpu_sc as plsc

_N = 10000
_E = 320000
_D = 128

_NC = 2
_NS = 16
_NW = _NC * _NS
_CH = 80
_NCHUNK = 125
_EPT = _CH * _NCHUNK
_ROWS = 10240
_RPS = _ROWS // _NS

def _zero_fill(buf, nrow):
    def row(i, _):
        for j in range(_D // 16):
            buf[i, pl.ds(j * 16, 16)] = jnp.zeros((16,), jnp.float32)
        return _
    lax.fori_loop(0, nrow, row, None)


def _deg_body(dst2d_hbm, out_hbm, dstv, ones_v, zb_v, acc, sem):
    c = lax.axis_index("c")
    s = lax.axis_index("s")
    wid = c * _NS + s
    last = wid == _NW - 1
    nrow = jnp.where(last, 20, 80)

    def fill_ones(i, _):
        ones_v[pl.ds(i * 16, 16)] = jnp.ones((16,), jnp.float32)
        return _
    lax.fori_loop(0, 128 // 16, fill_ones, None)

    def fill_zero(i, _):
        zb_v[pl.ds(i * 16, 16)] = jnp.zeros((16,), jnp.float32)
        return _
    lax.fori_loop(0, _RPS // 16, fill_zero, None)

    @pl.when(jnp.logical_not(last))
    def _load_full():
        pltpu.sync_copy(dst2d_hbm.at[pl.ds(wid * 80, 80)], dstv)

    @pl.when(last)
    def _load_tail():
        pltpu.sync_copy(dst2d_hbm.at[pl.ds(31 * 80, 20)], dstv.at[pl.ds(0, 20)])

    pltpu.sync_copy(zb_v, acc.at[pl.ds(s * _RPS, _RPS)])
    plsc.subcore_barrier()

    def issue(j, _):
        pltpu.async_copy(ones_v, acc.at[dstv.at[j]], sem, add=True)
        return _
    lax.fori_loop(0, nrow, issue, None)

    def drain(j, _):
        pltpu.make_async_copy(ones_v, acc.at[dstv.at[0]], sem).wait()
        return _
    lax.fori_loop(0, nrow, drain, None)

    plsc.subcore_barrier()
    pltpu.sync_copy(acc.at[pl.ds(s * _RPS, _RPS)],
                    out_hbm.at[c, pl.ds(s * _RPS, _RPS)])


def _agg_body(y_hbm, edges_hbm, out_hbm, *sc):
    rows = sc[0:4]
    srcb = sc[4:12]
    dstb = sc[12:20]
    acc = sc[20]
    gsem = sc[21:25]
    ssem = sc[25:29]
    lsem = sc[29:37]
    zsem = sc[37]

    c = lax.axis_index("c")
    s = lax.axis_index("s")
    wid = c * _NS + s
    base = wid * _EPT

    def load_start(j, w):
        pltpu.async_copy(edges_hbm.at[pl.ds(base + j * _CH, _CH)],
                         srcb[w], lsem[w])
        pltpu.async_copy(edges_hbm.at[pl.ds(_E + base + j * _CH, _CH)],
                         dstb[w], lsem[w])

    def load_wait(w):
        pltpu.make_async_copy(edges_hbm.at[pl.ds(base, _CH)],
                              srcb[w], lsem[w]).wait()
        pltpu.make_async_copy(edges_hbm.at[pl.ds(base, _CH)],
                              dstb[w], lsem[w]).wait()

    def gather_start(x, w):
        pltpu.async_copy(y_hbm.at[srcb[w]], rows[x], gsem[x])

    def gather_wait(x):
        pltpu.make_async_copy(y_hbm.at[srcb[0]], rows[x], gsem[x]).wait()

    def scat_start(x, w):
        pltpu.async_copy(rows[x], acc.at[dstb[w]], ssem[x], add=True)

    def scat_wait(x):
        pltpu.make_async_copy(rows[0], acc.at[dstb[0]], ssem[x]).wait()

    _zero_fill(rows[3], _CH)
    zd = [pltpu.async_copy(rows[3], acc.at[pl.ds(s * _RPS + i * _CH, _CH)], zsem)
          for i in range(_RPS // _CH)]
    for w in range(5):
        load_start(w, w)
    load_wait(0)
    gather_start(0, 0)
    load_wait(1)
    gather_start(1, 1)
    load_wait(2)
    gather_start(2, 2)
    for d in zd:
        d.wait()
    plsc.subcore_barrier()

    def step(j, t, s_wait=True, g_issue=True, l_issue=True):
        x = t % 4
        gather_wait(x)
        scat_start(x, t % 8)
        if g_issue:
            if s_wait:
                scat_wait((t + 3) % 4)
            load_wait((t + 3) % 8)
            gather_start((t + 3) % 4, (t + 3) % 8)
        if l_issue:
            load_start(j + 5, (t + 5) % 8)

    step(0, 0, s_wait=False)
    step(1, 1)
    step(2, 2)

    def group(g, _):
        for t8 in range(8):
            step(8 * g + 3 + t8, 3 + t8)
        return _
    lax.fori_loop(0, 14, group, None)

    for j in range(115, 120):
        step(j, j % 8)
    step(120, 120 % 8, l_issue=False)
    step(121, 121 % 8, l_issue=False)
    step(122, 122 % 8, g_issue=False, l_issue=False)
    step(123, 123 % 8, g_issue=False, l_issue=False)
    step(124, 124 % 8, g_issue=False, l_issue=False)
    for x in range(4):
        scat_wait((121 + x) % 4)

    plsc.subcore_barrier()
    pltpu.sync_copy(acc.at[pl.ds(s * _RPS, _RPS)],
                    out_hbm.at[c, pl.ds(s * _RPS, _RPS)])


@functools.lru_cache(maxsize=None)
def _sc_kernels():
    mesh = plsc.VectorSubcoreMesh(core_axis_name="c", subcore_axis_name="s",
                                  num_cores=_NC, num_subcores=_NS)
    deg = pl.kernel(
        _deg_body,
        out_type=jax.ShapeDtypeStruct((_NC, _ROWS), jnp.float32),
        mesh=mesh,
        scratch_types=[
            pltpu.VMEM((80, 128), jnp.int32),
            pltpu.VMEM((128,), jnp.float32),
            pltpu.VMEM((_RPS,), jnp.float32),
            pltpu.VMEM_SHARED((_ROWS,), jnp.float32),
            pltpu.SemaphoreType.DMA,
        ],
    )
    agg = pl.kernel(
        _agg_body,
        out_type=jax.ShapeDtypeStruct((_NC, _ROWS, _D), jnp.float32),
        mesh=mesh,
        scratch_types=(
            [pltpu.VMEM((_CH, _D), jnp.float32) for _ in range(4)]
            + [pltpu.VMEM((_CH,), jnp.int32) for _ in range(8)]
            + [pltpu.VMEM((_CH,), jnp.int32) for _ in range(8)]
            + [pltpu.VMEM_SHARED((_ROWS, _D), jnp.float32)]
            + [pltpu.SemaphoreType.DMA for _ in range(17)]
        ),
    )
    return deg, agg


def _dis_from(dp_ref):
    deg = dp_ref[0, :] + dp_ref[1, :] + 1.0
    return lax.rsqrt(deg)[:_N]


def _prep_body(x_ref, w_ref, dp_ref, y_ref):
    dis = _dis_from(dp_ref)
    xw = jnp.dot(x_ref[...], w_ref[...], preferred_element_type=jnp.float32)
    y_ref[...] = xw * dis[:, None]


def _mid_body(p_ref, y_ref, dp_ref, b_ref, w_ref, o_ref):
    dis = _dis_from(dp_ref)
    agg = p_ref[0, :_N, :] + p_ref[1, :_N, :]
    h = jnp.maximum(dis[:, None] * (agg + y_ref[...]) + b_ref[...][None, :], 0.0)
    hw = jnp.dot(h, w_ref[...], preferred_element_type=jnp.float32)
    o_ref[...] = hw * dis[:, None]


def _fin_body(p_ref, y_ref, dp_ref, b_ref, o_ref):
    dis = _dis_from(dp_ref)
    agg = p_ref[0, :_N, :] + p_ref[1, :_N, :]
    o_ref[...] = jnp.maximum(
        dis[:, None] * (agg + y_ref[...]) + b_ref[...][None, :], 0.0)


_f32 = jnp.float32
_prep_call = pl.pallas_call(
    _prep_body, out_shape=jax.ShapeDtypeStruct((_N, _D), _f32))
_mid_call = pl.pallas_call(
    _mid_body, out_shape=jax.ShapeDtypeStruct((_N, _D), _f32))
_fin_call = pl.pallas_call(
    _fin_body, out_shape=jax.ShapeDtypeStruct((_N, _D), _f32))


def kernel(x, edge_index, W1, b1, W2, b2):
    edges = edge_index.astype(jnp.int32).reshape(2 * _E)

    deg_kernel, agg_kernel = _sc_kernels()
    dst2d = edges[_E:].reshape(_E // 128, 128)
    dp = deg_kernel(dst2d)
    y1 = _prep_call(x, W1, dp)
    p1 = agg_kernel(y1, edges)
    y2 = _mid_call(p1, y1, dp, b1, W2)
    p2 = agg_kernel(y2, edges)
    return _fin_call(p2, y2, dp, b2)
